# Initial kernel scaffold; baseline (speedup 1.0000x reference)
import jax
import jax.numpy as jnp
from jax.experimental import pallas as pl


def kernel(ped_features, neigh_mask, k_emb, neigh_index, params):
    raise NotImplementedError("write your pallas kernel here")


# speedup vs baseline: 5.4769x; 5.4769x over previous
import functools

import jax
import jax.numpy as jnp
from jax import lax
from jax.experimental import pallas as pl
from jax.experimental.pallas import tpu as pltpu
from jax.experimental.pallas import tpu_sc as plsc

HID = 64
TBL_W = 80
ST_W = 8


def _silu(x):
    return x / (1.0 + jnp.exp(-x))


def _emb_body(pf, kemb, ew, eb, tbl_o, st_o):
    pfv = pf[...]
    p = pfv[:, 0:2]
    v = pfv[:, 2:4]
    a = pfv[:, 4:6]
    nv = jnp.sqrt(jnp.sum(v * v, axis=1, keepdims=True) + 1e-12)
    na = jnp.sqrt(jnp.sum(a * a, axis=1, keepdims=True) + 1e-12)
    hin = jnp.concatenate([nv, na, kemb[...]], axis=1)
    h0 = jnp.dot(hin, ew[...], preferred_element_type=jnp.float32) + eb[...]
    z = jnp.zeros((pfv.shape[0], TBL_W - HID - 2), jnp.float32)
    tbl_o[...] = jnp.concatenate([h0, p, z], axis=1)
    z2 = jnp.zeros((pfv.shape[0], 2), jnp.float32)
    st_o[...] = jnp.concatenate([p, v, a, z2], axis=1)


def _layer_body(bn, k, tbl, st, g, mask_r,
                w1h, w1n, w1d, b1, w2, b2,
                wx1, bx1, wx2r, bx2,
                wa1, ba1, wa2r, ba2,
                wh1a, wh1b, bh1, wh2, bh2,
                tbl_o, st_o):
    h = tbl[:, 0:HID]
    pos = tbl[:, HID:HID + 2]
    stv = st[...]
    vel = stv[:, 2:4]
    acce0 = stv[:, 4:6]

    g3 = g[...].reshape(bn, k, TBL_W)
    maskf = mask_r[...]
    mask3 = maskf != 0.0

    hn3 = jnp.where(mask3, g3[:, :, 0:HID], 0.0)
    rel3 = jnp.where(mask3, g3[:, :, HID:HID + 2] - pos[:, None, :], 0.0)
    d3 = jnp.sqrt(jnp.sum(rel3 * rel3, axis=2, keepdims=True) + 1e-12)

    hn2 = hn3.reshape(bn * k, HID)
    d2 = d3.reshape(bn * k, 1)

    e1self = jnp.dot(h, w1h[...], preferred_element_type=jnp.float32) + b1[...]
    e1self2 = jnp.broadcast_to(e1self[:, None, :], (bn, k, HID)).reshape(bn * k, HID)

    pre1 = (e1self2
            + jnp.dot(hn2, w1n[...], preferred_element_type=jnp.float32)
            + d2 * w1d[...])
    t1 = _silu(pre1)
    m2 = _silu(jnp.dot(t1, w2[...], preferred_element_type=jnp.float32) + b2[...])

    x1 = _silu(jnp.dot(m2, wx1[...], preferred_element_type=jnp.float32) + bx1[...])
    fx = jnp.sum(x1 * wx2r[...], axis=1, keepdims=True) + bx2[...]
    fx3 = fx.reshape(bn, k, 1)

    nn = jnp.sum(maskf, axis=1)
    agg = jnp.sum(rel3 * fx3, axis=1) / (nn + 1e-06)

    fah = _silu(jnp.dot(h, wa1[...], preferred_element_type=jnp.float32) + ba1[...])
    fa = jnp.sum(fah * wa2r[...], axis=1, keepdims=True) + ba2[...]

    a_new = fa * acce0 + agg
    v_new = vel + a_new
    x_new = pos + v_new

    m_i = jnp.sum(m2.reshape(bn, k, HID), axis=1)
    hh = _silu(jnp.dot(h, wh1a[...], preferred_element_type=jnp.float32)
               + jnp.dot(m_i, wh1b[...], preferred_element_type=jnp.float32)
               + bh1[...])
    h_new = h + jnp.dot(hh, wh2[...], preferred_element_type=jnp.float32) + bh2[...]

    z = jnp.zeros((bn, TBL_W - HID - 2), jnp.float32)
    tbl_o[...] = jnp.concatenate([h_new, x_new, z], axis=1)
    st_o[...] = jnp.concatenate([x_new, v_new, acce0, a_new], axis=1)


@functools.lru_cache(maxsize=None)
def _make_sc_gather(n_edges):
    info = plsc.get_sparse_core_info()
    nw = info.num_cores * info.num_subcores
    epw = n_edges // nw
    ch = 128
    nch = epw // ch
    mesh = plsc.VectorSubcoreMesh(core_axis_name="c", subcore_axis_name="s")

    @functools.partial(
        pl.kernel, mesh=mesh,
        compiler_params=pltpu.CompilerParams(use_tc_tiling_on_sc=False),
        out_type=jax.ShapeDtypeStruct((n_edges, TBL_W), jnp.float32),
        scratch_types=[
            pltpu.VMEM((ch,), jnp.int32),
            pltpu.VMEM((ch, TBL_W), jnp.float32),
            pltpu.SemaphoreType.DMA,
        ],
    )
    def gather(tbl_hbm, idx_hbm, out_hbm, idx_v, rows_v, sem):
        wid = lax.axis_index("s") * info.num_cores + lax.axis_index("c")
        base = wid * epw

        def body(i, carry):
            off = base + i * ch
            pltpu.sync_copy(idx_hbm.at[pl.ds(off, ch)], idx_v)
            pltpu.async_copy(tbl_hbm.at[idx_v], rows_v, sem).wait()
            pltpu.sync_copy(rows_v, out_hbm.at[pl.ds(off, ch)])
            return carry

        lax.fori_loop(0, nch, body, 0)

    return gather


def _sc_gather(tbl, idx_flat):
    return _make_sc_gather(idx_flat.shape[0])(tbl, idx_flat)


def kernel(ped_features, neigh_mask, k_emb, neigh_index, params):
    bs, N, k = neigh_index.shape
    nn_tot = bs * N
    ne = nn_tot * k
    bn = 256
    nb = nn_tot // bn

    pf2 = ped_features.reshape(nn_tot, 6)
    kemb2 = k_emb.reshape(nn_tot, 3)
    mask3d = neigh_mask.reshape(nn_tot, k, 1)
    idx_flat = (neigh_index.astype(jnp.int32)
                + (jnp.arange(bs, dtype=jnp.int32) * N)[:, None, None]).reshape(ne)

    tbl, st = pl.pallas_call(
        _emb_body,
        out_shape=[jax.ShapeDtypeStruct((nn_tot, TBL_W), jnp.float32),
                   jax.ShapeDtypeStruct((nn_tot, ST_W), jnp.float32)],
    )(pf2, kemb2, params['emb']['w'], params['emb']['b'].reshape(1, HID))

    full = lambda a: pl.BlockSpec(a.shape, lambda i: (0,) * a.ndim)

    for lp in params['layers']:
        w1 = lp['f_e1']['w']
        wh1 = lp['f_h1']['w']
        weights = [
            w1[0:HID], w1[HID:2 * HID], w1[2 * HID:2 * HID + 1],
            lp['f_e1']['b'].reshape(1, HID),
            lp['f_e2']['w'], lp['f_e2']['b'].reshape(1, HID),
            lp['f_x1']['w'], lp['f_x1']['b'].reshape(1, HID),
            lp['f_x2']['w'].reshape(1, HID), lp['f_x2']['b'].reshape(1, 1),
            lp['f_a1']['w'], lp['f_a1']['b'].reshape(1, HID),
            lp['f_a2']['w'].reshape(1, HID), lp['f_a2']['b'].reshape(1, 1),
            wh1[0:HID], wh1[HID:2 * HID], lp['f_h1']['b'].reshape(1, HID),
            lp['f_h2']['w'], lp['f_h2']['b'].reshape(1, HID),
        ]
        g = _sc_gather(tbl, idx_flat)
        tbl, st = pl.pallas_call(
            functools.partial(_layer_body, bn, k),
            grid=(nb,),
            in_specs=[
                pl.BlockSpec((bn, TBL_W), lambda i: (i, 0)),
                pl.BlockSpec((bn, ST_W), lambda i: (i, 0)),
                pl.BlockSpec((bn * k, TBL_W), lambda i: (i, 0)),
                pl.BlockSpec((bn, k, 1), lambda i: (i, 0, 0)),
            ] + [full(w) for w in weights],
            out_specs=[
                pl.BlockSpec((bn, TBL_W), lambda i: (i, 0)),
                pl.BlockSpec((bn, ST_W), lambda i: (i, 0)),
            ],
            out_shape=[jax.ShapeDtypeStruct((nn_tot, TBL_W), jnp.float32),
                       jax.ShapeDtypeStruct((nn_tot, ST_W), jnp.float32)],
        )(tbl, st, g, mask3d, *weights)

    return st[:, 6:8].reshape(bs, N, 2)


# speedup vs baseline: 5.7754x; 1.0545x over previous
import functools

import jax
import jax.numpy as jnp
from jax import lax
from jax.experimental import pallas as pl
from jax.experimental.pallas import tpu as pltpu
from jax.experimental.pallas import tpu_sc as plsc

HID = 64
TBL_W = 80
ST_W = 8


def _silu(x):
    return x / (1.0 + jnp.exp(-x))


def _emb_body(pf, kemb, ew, eb, tbl_o, st_o):
    pfv = pf[...]
    p = pfv[:, 0:2]
    v = pfv[:, 2:4]
    a = pfv[:, 4:6]
    nv = jnp.sqrt(jnp.sum(v * v, axis=1, keepdims=True) + 1e-12)
    na = jnp.sqrt(jnp.sum(a * a, axis=1, keepdims=True) + 1e-12)
    hin = jnp.concatenate([nv, na, kemb[...]], axis=1)
    h0 = jnp.dot(hin, ew[...], preferred_element_type=jnp.float32) + eb[...]
    z = jnp.zeros((pfv.shape[0], TBL_W - HID - 2), jnp.float32)
    tbl_o[...] = jnp.concatenate([h0, p, z], axis=1)
    z2 = jnp.zeros((pfv.shape[0], 2), jnp.float32)
    st_o[...] = jnp.concatenate([p, v, a, z2], axis=1)


def _layer_body(bn, k, tbl, st, g, mask_r,
                w1h, w1n, w1d, b1, w2, b2,
                wx1, bx1, wx2r, bx2,
                wa1, ba1, wa2r, ba2,
                wh1a, wh1b, bh1, wh2, bh2,
                tbl_o, st_o):
    h = tbl[:, 0:HID]
    pos = tbl[:, HID:HID + 2]
    stv = st[...]
    vel = stv[:, 2:4]
    acce0 = stv[:, 4:6]

    g3 = g[...].reshape(bn, k, TBL_W)
    maskf = mask_r[...]
    mask3 = maskf != 0.0

    hn3 = jnp.where(mask3, g3[:, :, 0:HID], 0.0)
    rel3 = jnp.where(mask3, g3[:, :, HID:HID + 2] - pos[:, None, :], 0.0)
    d3 = jnp.sqrt(jnp.sum(rel3 * rel3, axis=2, keepdims=True) + 1e-12)

    hn2 = hn3.reshape(bn * k, HID)
    d2 = d3.reshape(bn * k, 1)

    e1self = jnp.dot(h, w1h[...], preferred_element_type=jnp.float32) + b1[...]
    e1self2 = jnp.broadcast_to(e1self[:, None, :], (bn, k, HID)).reshape(bn * k, HID)

    pre1 = (e1self2
            + jnp.dot(hn2, w1n[...], preferred_element_type=jnp.float32)
            + d2 * w1d[...])
    t1 = _silu(pre1)
    m2 = _silu(jnp.dot(t1, w2[...], preferred_element_type=jnp.float32) + b2[...])

    x1 = _silu(jnp.dot(m2, wx1[...], preferred_element_type=jnp.float32) + bx1[...])
    fx = jnp.sum(x1 * wx2r[...], axis=1, keepdims=True) + bx2[...]
    fx3 = fx.reshape(bn, k, 1)

    nn = jnp.sum(maskf, axis=1)
    agg = jnp.sum(rel3 * fx3, axis=1) / (nn + 1e-06)

    fah = _silu(jnp.dot(h, wa1[...], preferred_element_type=jnp.float32) + ba1[...])
    fa = jnp.sum(fah * wa2r[...], axis=1, keepdims=True) + ba2[...]

    a_new = fa * acce0 + agg
    v_new = vel + a_new
    x_new = pos + v_new

    m_i = jnp.sum(m2.reshape(bn, k, HID), axis=1)
    hh = _silu(jnp.dot(h, wh1a[...], preferred_element_type=jnp.float32)
               + jnp.dot(m_i, wh1b[...], preferred_element_type=jnp.float32)
               + bh1[...])
    h_new = h + jnp.dot(hh, wh2[...], preferred_element_type=jnp.float32) + bh2[...]

    z = jnp.zeros((bn, TBL_W - HID - 2), jnp.float32)
    tbl_o[...] = jnp.concatenate([h_new, x_new, z], axis=1)
    st_o[...] = jnp.concatenate([x_new, v_new, acce0, a_new], axis=1)


@functools.lru_cache(maxsize=None)
def _make_sc_gather(n_edges):
    info = plsc.get_sparse_core_info()
    nw = info.num_cores * info.num_subcores
    epw = n_edges // nw
    ch = 128
    npair = epw // (2 * ch)
    mesh = plsc.VectorSubcoreMesh(core_axis_name="c", subcore_axis_name="s")

    @functools.partial(
        pl.kernel, mesh=mesh,
        compiler_params=pltpu.CompilerParams(use_tc_tiling_on_sc=False),
        out_type=jax.ShapeDtypeStruct((n_edges, TBL_W), jnp.float32),
        scratch_types=[
            pltpu.VMEM((2, ch), jnp.int32),
            pltpu.VMEM((2, ch, TBL_W), jnp.float32),
            pltpu.SemaphoreType.DMA,
            pltpu.SemaphoreType.DMA,
            pltpu.SemaphoreType.DMA,
            pltpu.SemaphoreType.DMA,
        ],
    )
    def gather(tbl_hbm, idx_hbm, out_hbm, idx_v, rows_v, gs0, gs1, ss0, ss1):
        wid = lax.axis_index("s") * info.num_cores + lax.axis_index("c")
        base = wid * epw

        def body(g, carry):
            c0 = base + 2 * g * ch
            c1 = c0 + ch
            pltpu.sync_copy(idx_hbm.at[pl.ds(c0, ch)], idx_v.at[0])
            h0 = pltpu.async_copy(tbl_hbm.at[idx_v.at[0]], rows_v.at[0], gs0)
            pltpu.sync_copy(idx_hbm.at[pl.ds(c1, ch)], idx_v.at[1])
            h1 = pltpu.async_copy(tbl_hbm.at[idx_v.at[1]], rows_v.at[1], gs1)
            h0.wait()
            s0 = pltpu.async_copy(rows_v.at[0], out_hbm.at[pl.ds(c0, ch)], ss0)
            h1.wait()
            s1 = pltpu.async_copy(rows_v.at[1], out_hbm.at[pl.ds(c1, ch)], ss1)
            s0.wait()
            s1.wait()
            return carry

        lax.fori_loop(0, npair, body, 0)

    return gather


def _sc_gather(tbl, idx_flat):
    return _make_sc_gather(idx_flat.shape[0])(tbl, idx_flat)


def kernel(ped_features, neigh_mask, k_emb, neigh_index, params):
    bs, N, k = neigh_index.shape
    nn_tot = bs * N
    ne = nn_tot * k
    bn = 256
    nb = nn_tot // bn

    pf2 = ped_features.reshape(nn_tot, 6)
    kemb2 = k_emb.reshape(nn_tot, 3)
    mask3d = neigh_mask.reshape(nn_tot, k, 1)
    idx_flat = (neigh_index.astype(jnp.int32)
                + (jnp.arange(bs, dtype=jnp.int32) * N)[:, None, None]).reshape(ne)

    tbl, st = pl.pallas_call(
        _emb_body,
        out_shape=[jax.ShapeDtypeStruct((nn_tot, TBL_W), jnp.float32),
                   jax.ShapeDtypeStruct((nn_tot, ST_W), jnp.float32)],
    )(pf2, kemb2, params['emb']['w'], params['emb']['b'].reshape(1, HID))

    full = lambda a: pl.BlockSpec(a.shape, lambda i: (0,) * a.ndim)

    for lp in params['layers']:
        w1 = lp['f_e1']['w']
        wh1 = lp['f_h1']['w']
        weights = [
            w1[0:HID], w1[HID:2 * HID], w1[2 * HID:2 * HID + 1],
            lp['f_e1']['b'].reshape(1, HID),
            lp['f_e2']['w'], lp['f_e2']['b'].reshape(1, HID),
            lp['f_x1']['w'], lp['f_x1']['b'].reshape(1, HID),
            lp['f_x2']['w'].reshape(1, HID), lp['f_x2']['b'].reshape(1, 1),
            lp['f_a1']['w'], lp['f_a1']['b'].reshape(1, HID),
            lp['f_a2']['w'].reshape(1, HID), lp['f_a2']['b'].reshape(1, 1),
            wh1[0:HID], wh1[HID:2 * HID], lp['f_h1']['b'].reshape(1, HID),
            lp['f_h2']['w'], lp['f_h2']['b'].reshape(1, HID),
        ]
        g = _sc_gather(tbl, idx_flat)
        tbl, st = pl.pallas_call(
            functools.partial(_layer_body, bn, k),
            grid=(nb,),
            in_specs=[
                pl.BlockSpec((bn, TBL_W), lambda i: (i, 0)),
                pl.BlockSpec((bn, ST_W), lambda i: (i, 0)),
                pl.BlockSpec((bn * k, TBL_W), lambda i: (i, 0)),
                pl.BlockSpec((bn, k, 1), lambda i: (i, 0, 0)),
            ] + [full(w) for w in weights],
            out_specs=[
                pl.BlockSpec((bn, TBL_W), lambda i: (i, 0)),
                pl.BlockSpec((bn, ST_W), lambda i: (i, 0)),
            ],
            out_shape=[jax.ShapeDtypeStruct((nn_tot, TBL_W), jnp.float32),
                       jax.ShapeDtypeStruct((nn_tot, ST_W), jnp.float32)],
        )(tbl, st, g, mask3d, *weights)

    return st[:, 6:8].reshape(bs, N, 2)


# speedup vs baseline: 5.8576x; 1.0142x over previous
import functools

import jax
import jax.numpy as jnp
from jax import lax
from jax.experimental import pallas as pl
from jax.experimental.pallas import tpu as pltpu
from jax.experimental.pallas import tpu_sc as plsc

HID = 64
TBL_W = 80
ST_W = 8


def _silu(x):
    return x / (1.0 + jnp.exp(-x))


def _emb_body(pf, kemb, ew, eb, tbl_o, st_o):
    pfv = pf[...]
    p = pfv[:, 0:2]
    v = pfv[:, 2:4]
    a = pfv[:, 4:6]
    nv = jnp.sqrt(jnp.sum(v * v, axis=1, keepdims=True) + 1e-12)
    na = jnp.sqrt(jnp.sum(a * a, axis=1, keepdims=True) + 1e-12)
    hin = jnp.concatenate([nv, na, kemb[...]], axis=1)
    h0 = jnp.dot(hin, ew[...], preferred_element_type=jnp.float32) + eb[...]
    z = jnp.zeros((pfv.shape[0], TBL_W - HID - 2), jnp.float32)
    tbl_o[...] = jnp.concatenate([h0, p, z], axis=1)
    z2 = jnp.zeros((pfv.shape[0], 2), jnp.float32)
    st_o[...] = jnp.concatenate([p, v, a, z2], axis=1)


def _layer_body(bn, k, tbl, st, g, mask_r,
                w1h, w1n, w1d, b1, w2, b2,
                wx1, bx1, wx2r, bx2,
                wa1, ba1, wa2r, ba2,
                wh1a, wh1b, bh1, wh2, bh2,
                tbl_o, st_o):
    h = tbl[:, 0:HID]
    pos = tbl[:, HID:HID + 2]
    stv = st[...]
    vel = stv[:, 2:4]
    acce0 = stv[:, 4:6]

    g3 = g[...].reshape(bn, k, TBL_W)
    maskf = mask_r[...]
    mask3 = maskf != 0.0

    hn3 = jnp.where(mask3, g3[:, :, 0:HID], 0.0)
    rel3 = jnp.where(mask3, g3[:, :, HID:HID + 2] - pos[:, None, :], 0.0)
    d3 = jnp.sqrt(jnp.sum(rel3 * rel3, axis=2, keepdims=True) + 1e-12)

    hn2 = hn3.reshape(bn * k, HID)
    d2 = d3.reshape(bn * k, 1)

    e1self = jnp.dot(h, w1h[...], preferred_element_type=jnp.float32) + b1[...]
    e1self2 = jnp.broadcast_to(e1self[:, None, :], (bn, k, HID)).reshape(bn * k, HID)

    pre1 = (e1self2
            + jnp.dot(hn2, w1n[...], preferred_element_type=jnp.float32)
            + d2 * w1d[...])
    t1 = _silu(pre1)
    m2 = _silu(jnp.dot(t1, w2[...], preferred_element_type=jnp.float32) + b2[...])

    x1 = _silu(jnp.dot(m2, wx1[...], preferred_element_type=jnp.float32) + bx1[...])
    fx = jnp.sum(x1 * wx2r[...], axis=1, keepdims=True) + bx2[...]
    fx3 = fx.reshape(bn, k, 1)

    nn = jnp.sum(maskf, axis=1)
    agg = jnp.sum(rel3 * fx3, axis=1) / (nn + 1e-06)

    fah = _silu(jnp.dot(h, wa1[...], preferred_element_type=jnp.float32) + ba1[...])
    fa = jnp.sum(fah * wa2r[...], axis=1, keepdims=True) + ba2[...]

    a_new = fa * acce0 + agg
    v_new = vel + a_new
    x_new = pos + v_new

    m_i = jnp.sum(m2.reshape(bn, k, HID), axis=1)
    hh = _silu(jnp.dot(h, wh1a[...], preferred_element_type=jnp.float32)
               + jnp.dot(m_i, wh1b[...], preferred_element_type=jnp.float32)
               + bh1[...])
    h_new = h + jnp.dot(hh, wh2[...], preferred_element_type=jnp.float32) + bh2[...]

    z = jnp.zeros((bn, TBL_W - HID - 2), jnp.float32)
    tbl_o[...] = jnp.concatenate([h_new, x_new, z], axis=1)
    st_o[...] = jnp.concatenate([x_new, v_new, acce0, a_new], axis=1)


@functools.lru_cache(maxsize=None)
def _make_sc_gather(n_edges):
    info = plsc.get_sparse_core_info()
    nw = info.num_cores * info.num_subcores
    epw = n_edges // nw
    ch = 128
    nbuf = 4
    ngrp = epw // (nbuf * ch)
    mesh = plsc.VectorSubcoreMesh(core_axis_name="c", subcore_axis_name="s")

    @functools.partial(
        pl.kernel, mesh=mesh,
        compiler_params=pltpu.CompilerParams(use_tc_tiling_on_sc=False),
        out_type=jax.ShapeDtypeStruct((n_edges, TBL_W), jnp.float32),
        scratch_types=[
            pltpu.VMEM((epw,), jnp.int32),
            pltpu.VMEM((nbuf, ch, TBL_W), jnp.float32),
        ] + [pltpu.SemaphoreType.DMA] * (2 * nbuf),
    )
    def gather(tbl_hbm, idx_hbm, out_hbm, idx_v, rows_v, *sems):
        gs = sems[:nbuf]
        ss = sems[nbuf:]
        wid = lax.axis_index("s") * info.num_cores + lax.axis_index("c")
        base = wid * epw
        pltpu.sync_copy(idx_hbm.at[pl.ds(base, epw)], idx_v)

        def body(g, carry):
            off = g * nbuf * ch
            hs = []
            for b in range(nbuf):
                hs.append(pltpu.async_copy(
                    tbl_hbm.at[idx_v.at[pl.ds(off + b * ch, ch)]],
                    rows_v.at[b], gs[b]))
            sts = []
            for b in range(nbuf):
                hs[b].wait()
                sts.append(pltpu.async_copy(
                    rows_v.at[b], out_hbm.at[pl.ds(base + off + b * ch, ch)],
                    ss[b]))
            for b in range(nbuf):
                sts[b].wait()
            return carry

        lax.fori_loop(0, ngrp, body, 0)

    return gather


def _sc_gather(tbl, idx_flat):
    return _make_sc_gather(idx_flat.shape[0])(tbl, idx_flat)


def kernel(ped_features, neigh_mask, k_emb, neigh_index, params):
    bs, N, k = neigh_index.shape
    nn_tot = bs * N
    ne = nn_tot * k
    bn = 256
    nb = nn_tot // bn

    pf2 = ped_features.reshape(nn_tot, 6)
    kemb2 = k_emb.reshape(nn_tot, 3)
    mask3d = neigh_mask.reshape(nn_tot, k, 1)
    idx_flat = (neigh_index.astype(jnp.int32)
                + (jnp.arange(bs, dtype=jnp.int32) * N)[:, None, None]).reshape(ne)

    tbl, st = pl.pallas_call(
        _emb_body,
        out_shape=[jax.ShapeDtypeStruct((nn_tot, TBL_W), jnp.float32),
                   jax.ShapeDtypeStruct((nn_tot, ST_W), jnp.float32)],
    )(pf2, kemb2, params['emb']['w'], params['emb']['b'].reshape(1, HID))

    full = lambda a: pl.BlockSpec(a.shape, lambda i: (0,) * a.ndim)

    for lp in params['layers']:
        w1 = lp['f_e1']['w']
        wh1 = lp['f_h1']['w']
        weights = [
            w1[0:HID], w1[HID:2 * HID], w1[2 * HID:2 * HID + 1],
            lp['f_e1']['b'].reshape(1, HID),
            lp['f_e2']['w'], lp['f_e2']['b'].reshape(1, HID),
            lp['f_x1']['w'], lp['f_x1']['b'].reshape(1, HID),
            lp['f_x2']['w'].reshape(1, HID), lp['f_x2']['b'].reshape(1, 1),
            lp['f_a1']['w'], lp['f_a1']['b'].reshape(1, HID),
            lp['f_a2']['w'].reshape(1, HID), lp['f_a2']['b'].reshape(1, 1),
            wh1[0:HID], wh1[HID:2 * HID], lp['f_h1']['b'].reshape(1, HID),
            lp['f_h2']['w'], lp['f_h2']['b'].reshape(1, HID),
        ]
        g = _sc_gather(tbl, idx_flat)
        tbl, st = pl.pallas_call(
            functools.partial(_layer_body, bn, k),
            grid=(nb,),
            in_specs=[
                pl.BlockSpec((bn, TBL_W), lambda i: (i, 0)),
                pl.BlockSpec((bn, ST_W), lambda i: (i, 0)),
                pl.BlockSpec((bn * k, TBL_W), lambda i: (i, 0)),
                pl.BlockSpec((bn, k, 1), lambda i: (i, 0, 0)),
            ] + [full(w) for w in weights],
            out_specs=[
                pl.BlockSpec((bn, TBL_W), lambda i: (i, 0)),
                pl.BlockSpec((bn, ST_W), lambda i: (i, 0)),
            ],
            out_shape=[jax.ShapeDtypeStruct((nn_tot, TBL_W), jnp.float32),
                       jax.ShapeDtypeStruct((nn_tot, ST_W), jnp.float32)],
        )(tbl, st, g, mask3d, *weights)

    return st[:, 6:8].reshape(bs, N, 2)


# speedup vs baseline: 7.0381x; 1.2015x over previous
import functools

import jax
import jax.numpy as jnp
from jax import lax
from jax.experimental import pallas as pl
from jax.experimental.pallas import tpu as pltpu
from jax.experimental.pallas import tpu_sc as plsc

HID = 64
TBL_W = 80
ST_W = 8


def _silu(x):
    return x * jax.nn.sigmoid(x)


def _dot(a, b):
    return lax.dot_general(
        a.astype(jnp.bfloat16), b.astype(jnp.bfloat16),
        ((( a.ndim - 1,), (0,)), ((), ())),
        preferred_element_type=jnp.float32)


def _emb_body(pf, kemb, mask, ew, eb, tbl_o, st_o):
    pfv = pf[...]
    p = pfv[:, 0:2]
    v = pfv[:, 2:4]
    a = pfv[:, 4:6]
    nv = jnp.sqrt(jnp.sum(v * v, axis=1, keepdims=True) + 1e-12)
    na = jnp.sqrt(jnp.sum(a * a, axis=1, keepdims=True) + 1e-12)
    hin = jnp.concatenate([nv, na, kemb[...]], axis=1)
    h0 = _dot(hin, ew[...]) + eb[...]
    nn_eps = jnp.sum(mask[...], axis=1, keepdims=True) + 1e-06
    n = pfv.shape[0]
    z = jnp.zeros((n, TBL_W - HID - 3), jnp.float32)
    zh = jnp.zeros((n, HID), jnp.float32)
    top = jnp.concatenate([h0, p, nn_eps, z], axis=1)
    bot = jnp.concatenate([zh, p, nn_eps, z], axis=1)
    tbl_o[...] = jnp.concatenate([top[None], bot[None]], axis=0)
    z2 = jnp.zeros((n, 2), jnp.float32)
    st_o[...] = jnp.concatenate([p, v, a, z2], axis=1)


def _layer_body(bn, k, tbl, st, g,
                w1h, w1n, w1d, b1, w2, b2,
                wx1, bx1, wx2r, bx2,
                wa1, ba1, wa2r, ba2,
                wh1a, wh1b, bh1, wh2, bh2,
                tbl_o, st_o):
    tblv = tbl[...][0]
    h = tblv[:, 0:HID]
    pos = tblv[:, HID:HID + 2]
    nn_eps = tblv[:, HID + 2:HID + 3]
    stv = st[...]
    vel = stv[:, 2:4]
    acce0 = stv[:, 4:6]

    g3 = g[...].reshape(bn, k, TBL_W)
    hn3 = g3[:, :, 0:HID]
    rel3 = g3[:, :, HID:HID + 2] - pos[:, None, :]
    rx = rel3[:, :, 0:1]
    ry = rel3[:, :, 1:2]
    d3 = jnp.sqrt(rx * rx + ry * ry + 1e-12)

    hn2 = hn3.reshape(bn * k, HID)
    d2 = d3.reshape(bn * k, 1)

    e1self = _dot(h, w1h[...]) + b1[...]
    e1self2 = jnp.broadcast_to(e1self[:, None, :], (bn, k, HID)).reshape(bn * k, HID)

    d2b = d2.astype(jnp.bfloat16).astype(jnp.float32)
    pre1 = (e1self2
            + _dot(hn2, w1n[...])
            + d2b * w1d[...].astype(jnp.bfloat16).astype(jnp.float32))
    t1 = _silu(pre1)
    m2 = _silu(_dot(t1, w2[...]) + b2[...])

    x1 = _silu(_dot(m2, wx1[...]) + bx1[...])
    fx = _dot(x1, wx2r[...]) + bx2[...]
    fx3 = fx.reshape(bn, k, 1)

    agg = jnp.sum(rel3 * fx3, axis=1) / nn_eps

    fah = _silu(_dot(h, wa1[...]) + ba1[...])
    fa = _dot(fah, wa2r[...]) + ba2[...]

    a_new = fa * acce0 + agg
    v_new = vel + a_new
    x_new = pos + v_new

    m_i = jnp.sum(m2.reshape(bn, k, HID), axis=1)
    hh = _silu(_dot(h, wh1a[...]) + _dot(m_i, wh1b[...]) + bh1[...])
    h_new = h + _dot(hh, wh2[...]) + bh2[...]

    z = jnp.zeros((bn, TBL_W - HID - 3), jnp.float32)
    zh = jnp.zeros((bn, HID), jnp.float32)
    top = jnp.concatenate([h_new, x_new, nn_eps, z], axis=1)
    bot = jnp.concatenate([zh, x_new, nn_eps, z], axis=1)
    tbl_o[...] = jnp.concatenate([top[None], bot[None]], axis=0)
    st_o[...] = jnp.concatenate([x_new, v_new, acce0, a_new], axis=1)


@functools.lru_cache(maxsize=None)
def _make_sc_gather(n_edges, n_rows):
    info = plsc.get_sparse_core_info()
    nw = info.num_cores * info.num_subcores
    epw = n_edges // nw
    ch = 128
    nbuf = 4
    ngrp = epw // (nbuf * ch)
    mesh = plsc.VectorSubcoreMesh(core_axis_name="c", subcore_axis_name="s")

    @functools.partial(
        pl.kernel, mesh=mesh,
        compiler_params=pltpu.CompilerParams(use_tc_tiling_on_sc=False),
        out_type=jax.ShapeDtypeStruct((n_edges, TBL_W), jnp.float32),
        scratch_types=[
            pltpu.VMEM((epw,), jnp.int32),
            pltpu.VMEM((nbuf, ch, TBL_W), jnp.float32),
        ] + [pltpu.SemaphoreType.DMA] * (2 * nbuf),
    )
    def gather(tbl_hbm, idx_hbm, out_hbm, idx_v, rows_v, *sems):
        gs = sems[:nbuf]
        ss = sems[nbuf:]
        wid = lax.axis_index("s") * info.num_cores + lax.axis_index("c")
        base = wid * epw
        pltpu.sync_copy(idx_hbm.at[pl.ds(base, epw)], idx_v)

        def body(g, carry):
            off = g * nbuf * ch
            hs = []
            for b in range(nbuf):
                hs.append(pltpu.async_copy(
                    tbl_hbm.at[idx_v.at[pl.ds(off + b * ch, ch)]],
                    rows_v.at[b], gs[b]))
            sts = []
            for b in range(nbuf):
                hs[b].wait()
                sts.append(pltpu.async_copy(
                    rows_v.at[b], out_hbm.at[pl.ds(base + off + b * ch, ch)],
                    ss[b]))
            for b in range(nbuf):
                sts[b].wait()
            return carry

        lax.fori_loop(0, ngrp, body, 0)

    return gather


def _sc_gather(tbl2, idx_flat):
    flat = tbl2.reshape(tbl2.shape[0] * tbl2.shape[1], TBL_W)
    return _make_sc_gather(idx_flat.shape[0], flat.shape[0])(flat, idx_flat)


def kernel(ped_features, neigh_mask, k_emb, neigh_index, params):
    bs, N, k = neigh_index.shape
    nn_tot = bs * N
    ne = nn_tot * k
    bn = 256
    nb = nn_tot // bn

    pf2 = ped_features.reshape(nn_tot, 6)
    kemb2 = k_emb.reshape(nn_tot, 3)
    mask2 = neigh_mask.reshape(nn_tot, k)
    gid = (neigh_index.astype(jnp.int32)
           + (jnp.arange(bs, dtype=jnp.int32) * N)[:, None, None])
    self_gid = jnp.arange(nn_tot, dtype=jnp.int32).reshape(bs, N, 1) + nn_tot
    idx_flat = jnp.where(neigh_mask != 0.0, gid, self_gid).reshape(ne)

    tbl, st = pl.pallas_call(
        _emb_body,
        out_shape=[jax.ShapeDtypeStruct((2, nn_tot, TBL_W), jnp.float32),
                   jax.ShapeDtypeStruct((nn_tot, ST_W), jnp.float32)],
    )(pf2, kemb2, mask2, params['emb']['w'], params['emb']['b'].reshape(1, HID))

    full = lambda a: pl.BlockSpec(a.shape, lambda i: (0,) * a.ndim)

    for lp in params['layers']:
        w1 = lp['f_e1']['w']
        wh1 = lp['f_h1']['w']
        weights = [
            w1[0:HID], w1[HID:2 * HID], w1[2 * HID:2 * HID + 1],
            lp['f_e1']['b'].reshape(1, HID),
            lp['f_e2']['w'], lp['f_e2']['b'].reshape(1, HID),
            lp['f_x1']['w'], lp['f_x1']['b'].reshape(1, HID),
            lp['f_x2']['w'], lp['f_x2']['b'].reshape(1, 1),
            lp['f_a1']['w'], lp['f_a1']['b'].reshape(1, HID),
            lp['f_a2']['w'], lp['f_a2']['b'].reshape(1, 1),
            wh1[0:HID], wh1[HID:2 * HID], lp['f_h1']['b'].reshape(1, HID),
            lp['f_h2']['w'], lp['f_h2']['b'].reshape(1, HID),
        ]
        g = _sc_gather(tbl, idx_flat)
        tbl, st = pl.pallas_call(
            functools.partial(_layer_body, bn, k),
            grid=(nb,),
            in_specs=[
                pl.BlockSpec((1, bn, TBL_W), lambda i: (0, i, 0)),
                pl.BlockSpec((bn, ST_W), lambda i: (i, 0)),
                pl.BlockSpec((bn * k, TBL_W), lambda i: (i, 0)),
            ] + [full(w) for w in weights],
            out_specs=[
                pl.BlockSpec((2, bn, TBL_W), lambda i: (0, i, 0)),
                pl.BlockSpec((bn, ST_W), lambda i: (i, 0)),
            ],
            out_shape=[jax.ShapeDtypeStruct((2, nn_tot, TBL_W), jnp.float32),
                       jax.ShapeDtypeStruct((nn_tot, ST_W), jnp.float32)],
        )(tbl, st, g, *weights)

    return st[:, 6:8].reshape(bs, N, 2)


# speedup vs baseline: 7.4202x; 1.0543x over previous
import functools

import jax
import jax.numpy as jnp
from jax import lax
from jax.experimental import pallas as pl
from jax.experimental.pallas import tpu as pltpu
from jax.experimental.pallas import tpu_sc as plsc

HID = 64
TBL_W = 80
ST_W = 8


def _silu(x):
    return x / (1.0 + jnp.exp(-x))


def _dot(a, b):
    return lax.dot_general(
        a.astype(jnp.bfloat16), b.astype(jnp.bfloat16),
        ((( a.ndim - 1,), (0,)), ((), ())),
        preferred_element_type=jnp.float32)


def _emb_body(pf, kemb, mask, ew, eb, tbl_o, st_o):
    pfv = pf[...]
    p = pfv[:, 0:2]
    v = pfv[:, 2:4]
    a = pfv[:, 4:6]
    nv = jnp.sqrt(jnp.sum(v * v, axis=1, keepdims=True) + 1e-12)
    na = jnp.sqrt(jnp.sum(a * a, axis=1, keepdims=True) + 1e-12)
    hin = jnp.concatenate([nv, na, kemb[...]], axis=1)
    h0 = _dot(hin, ew[...]) + eb[...]
    nn_eps = jnp.sum(mask[...], axis=1, keepdims=True) + 1e-06
    n = pfv.shape[0]
    z = jnp.zeros((n, TBL_W - HID - 3), jnp.float32)
    zh = jnp.zeros((n, HID), jnp.float32)
    top = jnp.concatenate([h0, p, nn_eps, z], axis=1)
    bot = jnp.concatenate([zh, p, nn_eps, z], axis=1)
    tbl_o[...] = jnp.concatenate([top[None], bot[None]], axis=0)
    z2 = jnp.zeros((n, 2), jnp.float32)
    st_o[...] = jnp.concatenate([p, v, a, z2], axis=1)


def _layer_body(bn, k, tbl, st, g,
                w1h, w1n, w1d, b1, w2, b2,
                wx1, bx1, wx2r, bx2,
                wa1, ba1, wa2r, ba2,
                wh1a, wh1b, bh1, wh2, bh2,
                tbl_o, st_o):
    tblv = tbl[...][0]
    h = tblv[:, 0:HID]
    pos = tblv[:, HID:HID + 2]
    nn_eps = tblv[:, HID + 2:HID + 3]
    stv = st[...]
    vel = stv[:, 2:4]
    acce0 = stv[:, 4:6]

    g3 = g[...].reshape(bn, k, TBL_W)
    hn3 = g3[:, :, 0:HID]
    rel3 = g3[:, :, HID:HID + 2] - pos[:, None, :]
    rx = rel3[:, :, 0:1]
    ry = rel3[:, :, 1:2]
    d3 = jnp.sqrt(rx * rx + ry * ry + 1e-12)

    hn2 = hn3.reshape(bn * k, HID)
    d2 = d3.reshape(bn * k, 1)

    e1self = _dot(h, w1h[...]) + b1[...]
    e1self2 = jnp.broadcast_to(e1self[:, None, :], (bn, k, HID)).reshape(bn * k, HID)

    hd = jnp.concatenate([hn2, d2], axis=1)
    w1nd = jnp.concatenate([w1n[...], w1d[...]], axis=0)
    pre1 = e1self2 + _dot(hd, w1nd)
    t1 = _silu(pre1)
    m2 = _silu(_dot(t1, w2[...]) + b2[...])

    x1 = _silu(_dot(m2, wx1[...]) + bx1[...])
    fx = _dot(x1, wx2r[...]) + bx2[...]
    fx3 = fx.reshape(bn, k, 1)

    agg = jnp.sum(rel3 * fx3, axis=1) / nn_eps

    fah = _silu(_dot(h, wa1[...]) + ba1[...])
    fa = _dot(fah, wa2r[...]) + ba2[...]

    a_new = fa * acce0 + agg
    v_new = vel + a_new
    x_new = pos + v_new

    m_i = jnp.sum(m2.reshape(bn, k, HID), axis=1)
    hh = _silu(_dot(h, wh1a[...]) + _dot(m_i, wh1b[...]) + bh1[...])
    h_new = h + _dot(hh, wh2[...]) + bh2[...]

    z = jnp.zeros((bn, TBL_W - HID - 3), jnp.float32)
    zh = jnp.zeros((bn, HID), jnp.float32)
    top = jnp.concatenate([h_new, x_new, nn_eps, z], axis=1)
    bot = jnp.concatenate([zh, x_new, nn_eps, z], axis=1)
    tbl_o[...] = jnp.concatenate([top[None], bot[None]], axis=0)
    st_o[...] = jnp.concatenate([x_new, v_new, acce0, a_new], axis=1)


@functools.lru_cache(maxsize=None)
def _make_sc_gather(n_edges, n_rows):
    info = plsc.get_sparse_core_info()
    nw = info.num_cores * info.num_subcores
    epw = n_edges // nw
    ch = 128
    nbuf = 4
    ngrp = epw // (nbuf * ch)
    mesh = plsc.VectorSubcoreMesh(core_axis_name="c", subcore_axis_name="s")

    @functools.partial(
        pl.kernel, mesh=mesh,
        compiler_params=pltpu.CompilerParams(use_tc_tiling_on_sc=False),
        out_type=jax.ShapeDtypeStruct((n_edges, TBL_W), jnp.float32),
        scratch_types=[
            pltpu.VMEM((epw,), jnp.int32),
            pltpu.VMEM((nbuf, ch, TBL_W), jnp.float32),
        ] + [pltpu.SemaphoreType.DMA] * (2 * nbuf),
    )
    def gather(tbl_hbm, idx_hbm, out_hbm, idx_v, rows_v, *sems):
        gs = sems[:nbuf]
        ss = sems[nbuf:]
        wid = lax.axis_index("s") * info.num_cores + lax.axis_index("c")
        base = wid * epw
        pltpu.sync_copy(idx_hbm.at[pl.ds(base, epw)], idx_v)

        def body(g, carry):
            off = g * nbuf * ch
            hs = []
            for b in range(nbuf):
                hs.append(pltpu.async_copy(
                    tbl_hbm.at[idx_v.at[pl.ds(off + b * ch, ch)]],
                    rows_v.at[b], gs[b]))
            sts = []
            for b in range(nbuf):
                hs[b].wait()
                sts.append(pltpu.async_copy(
                    rows_v.at[b], out_hbm.at[pl.ds(base + off + b * ch, ch)],
                    ss[b]))
            for b in range(nbuf):
                sts[b].wait()
            return carry

        lax.fori_loop(0, ngrp, body, 0)

    return gather


def _sc_gather(tbl2, idx_flat):
    flat = tbl2.reshape(tbl2.shape[0] * tbl2.shape[1], TBL_W)
    return _make_sc_gather(idx_flat.shape[0], flat.shape[0])(flat, idx_flat)


def kernel(ped_features, neigh_mask, k_emb, neigh_index, params):
    bs, N, k = neigh_index.shape
    nn_tot = bs * N
    ne = nn_tot * k
    bn = 256
    nb = nn_tot // bn

    pf2 = ped_features.reshape(nn_tot, 6)
    kemb2 = k_emb.reshape(nn_tot, 3)
    mask2 = neigh_mask.reshape(nn_tot, k)
    gid = (neigh_index.astype(jnp.int32)
           + (jnp.arange(bs, dtype=jnp.int32) * N)[:, None, None])
    self_gid = jnp.arange(nn_tot, dtype=jnp.int32).reshape(bs, N, 1) + nn_tot
    idx_flat = jnp.where(neigh_mask != 0.0, gid, self_gid).reshape(ne)

    tbl, st = pl.pallas_call(
        _emb_body,
        out_shape=[jax.ShapeDtypeStruct((2, nn_tot, TBL_W), jnp.float32),
                   jax.ShapeDtypeStruct((nn_tot, ST_W), jnp.float32)],
    )(pf2, kemb2, mask2, params['emb']['w'], params['emb']['b'].reshape(1, HID))

    full = lambda a: pl.BlockSpec(a.shape, lambda i: (0,) * a.ndim)

    for lp in params['layers']:
        w1 = lp['f_e1']['w']
        wh1 = lp['f_h1']['w']
        weights = [
            w1[0:HID], w1[HID:2 * HID], w1[2 * HID:2 * HID + 1],
            lp['f_e1']['b'].reshape(1, HID),
            lp['f_e2']['w'], lp['f_e2']['b'].reshape(1, HID),
            lp['f_x1']['w'], lp['f_x1']['b'].reshape(1, HID),
            lp['f_x2']['w'], lp['f_x2']['b'].reshape(1, 1),
            lp['f_a1']['w'], lp['f_a1']['b'].reshape(1, HID),
            lp['f_a2']['w'], lp['f_a2']['b'].reshape(1, 1),
            wh1[0:HID], wh1[HID:2 * HID], lp['f_h1']['b'].reshape(1, HID),
            lp['f_h2']['w'], lp['f_h2']['b'].reshape(1, HID),
        ]
        g = _sc_gather(tbl, idx_flat)
        tbl, st = pl.pallas_call(
            functools.partial(_layer_body, bn, k),
            grid=(nb,),
            in_specs=[
                pl.BlockSpec((1, bn, TBL_W), lambda i: (0, i, 0)),
                pl.BlockSpec((bn, ST_W), lambda i: (i, 0)),
                pl.BlockSpec((bn * k, TBL_W), lambda i: (i, 0)),
            ] + [full(w) for w in weights],
            out_specs=[
                pl.BlockSpec((2, bn, TBL_W), lambda i: (0, i, 0)),
                pl.BlockSpec((bn, ST_W), lambda i: (i, 0)),
            ],
            out_shape=[jax.ShapeDtypeStruct((2, nn_tot, TBL_W), jnp.float32),
                       jax.ShapeDtypeStruct((nn_tot, ST_W), jnp.float32)],
        )(tbl, st, g, *weights)

    return st[:, 6:8].reshape(bs, N, 2)
